# baseline (speedup 1.0000x reference)
import functools

import jax
import jax.numpy as jnp
from jax import lax
from jax.experimental import pallas as pl
from jax.experimental.pallas import tpu as pltpu
from jax.experimental.pallas import tpu_sc as plsc

VOCAB = 1000000
DIM = 64
B, L = 4096, 200
NTOK = B * L

NC, NS = 2, 16
NW = NC * NS
CB = 64
PER_W = NTOK // NW
NCHUNK = PER_W // CB
NBUF = 8
LOOK = 7
NGROUP = NCHUNK // NBUF


def _sc_gather(idx3, table):
    mesh = plsc.VectorSubcoreMesh(core_axis_name="c", subcore_axis_name="s")

    @functools.partial(
        pl.kernel,
        mesh=mesh,
        compiler_params=pltpu.CompilerParams(use_tc_tiling_on_sc=False),
        out_type=jax.ShapeDtypeStruct((NTOK, 128), jnp.float32),
        scratch_types=[
            pltpu.VMEM((NCHUNK, CB), jnp.int32),
            pltpu.VMEM((NBUF, CB, 128), jnp.float32),
        ] + [pltpu.SemaphoreType.DMA] * (2 * NBUF),
    )
    def k(idx_hbm, w_hbm, out_hbm, idx_v, rows_v, *sems):
        sem_g, sem_p = sems[:NBUF], sems[NBUF:]
        wid = lax.axis_index("s") * NC + lax.axis_index("c")
        base = wid * PER_W
        pltpu.sync_copy(idx_hbm.at[wid], idx_v)

        def gather(j, b):
            pltpu.async_copy(w_hbm.at[idx_v.at[j]], rows_v.at[b], sem_g[b])

        def wait_gather(j, b):
            pltpu.make_async_copy(
                w_hbm.at[idx_v.at[j]], rows_v.at[b], sem_g[b]).wait()

        def put(j, b):
            pltpu.async_copy(
                rows_v.at[b, :, pl.ds(0, DIM)],
                out_hbm.at[pl.ds(base + j * CB, CB), pl.ds(0, DIM)], sem_p[b])

        def wait_put(b):
            pltpu.make_async_copy(
                rows_v.at[b, :, pl.ds(0, DIM)],
                out_hbm.at[pl.ds(base, CB), pl.ds(0, DIM)], sem_p[b]).wait()

        for t in range(LOOK):
            gather(t, t)
        for b in range(NBUF):
            wait_gather(b, b)
            put(b, b)
            jn = b + LOOK
            bn = jn % NBUF
            if jn >= NBUF:
                wait_put(bn)
            gather(jn, bn)

        def group(g, c):
            j0 = g * NBUF
            for b in range(NBUF):
                j = j0 + b
                wait_gather(j, b)
                put(j, b)
                bn = (b + LOOK) % NBUF
                wait_put(bn)
                gather(j + LOOK, bn)
            return c

        lax.fori_loop(1, NGROUP - 1, group, 0)

        j0 = NCHUNK - NBUF
        for b in range(NBUF):
            j = j0 + b
            wait_gather(j, b)
            put(j, b)
            jn = j + LOOK
            if jn < NCHUNK:
                bn = (b + LOOK) % NBUF
                wait_put(bn)
                gather(jn, bn)
        for b in range(NBUF):
            wait_put(b)

    return k(idx3, table)


def kernel(input_word, W):
    idx3 = input_word.reshape(NW, NCHUNK, CB)
    Wp = jnp.pad(W, ((0, 0), (0, 128 - DIM)))
    out = _sc_gather(idx3, Wp)
    return out[:, :DIM].reshape(B, L, DIM)
